# baseline (speedup 1.0000x reference)
import jax
import jax.numpy as jnp
from jax.experimental import pallas as pl
from jax.experimental.pallas import tpu as pltpu

BLOCK_R = 512
WINDOW = 64
NEG = -1e30
QSCALE = (128 ** -0.5) * 1.4426950408889634


def _dotT(a, w):
    return jax.lax.dot_general(
        a, w, (((1,), (1,)), ((), ())), preferred_element_type=jnp.float32
    )


def _make_body(n_blk, last):
    def _fused_body(emb_ref, prev_ref, ww_ref, bw_ref, wq_ref, bq_ref,
                    wo_ref, bo_ref, out_ref, p_s, keys_s, denom_s):
        g = pl.program_id(0)

        keys_prev = keys_s[...]
        p_prev = p_s[...]
        ret = jax.lax.dot_general(
            p_prev, keys_prev, (((1,), (0,)), ((), ())),
            preferred_element_type=jnp.float32) / denom_s[...]
        out = _dotT(ret, wo_ref[...]) + bo_ref[...]
        out_ref[0] = out

        @pl.when((g - 1) % n_blk == 0)
        def _zero_t0():
            out_ref[0, 0:1, :] = jnp.zeros((1, out.shape[1]), jnp.float32)

        a_i = jnp.minimum(g, last) % n_blk
        e = emb_ref[0]
        ep = prev_ref[0]

        q = (_dotT(e, wq_ref[...]) + bq_ref[...]) * QSCALE
        k_cur = _dotT(e, ww_ref[...]) + bw_ref[...]
        k_prev = _dotT(ep, ww_ref[...]) + bw_ref[...]
        keys = jnp.concatenate([k_prev, k_cur], axis=0)

        sim = _dotT(q, keys)
        rows = jax.lax.broadcasted_iota(jnp.int32, sim.shape, 0)
        cols = jax.lax.broadcasted_iota(jnp.int32, sim.shape, 1)
        valid = (cols >= rows) & (cols <= rows + WINDOW - 1) & \
            ((cols >= WINDOW) | (a_i > 0))
        sim = jnp.where(valid, sim, NEG)

        p = jnp.exp2(sim)
        denom_s[...] = jnp.sum(p, axis=1, keepdims=True)
        p_s[...] = p
        keys_s[...] = keys

    return _fused_body


def kernel(embeddings, W_write, b_write, W_read_q, b_read_q, W_out, b_out):
    B, T, E = embeddings.shape
    D = W_write.shape[0]
    R, W = BLOCK_R, WINDOW
    n_blk = T // R
    G = B * n_blk + 1
    last = B * n_blk - 1

    def a_map(g):
        idx = jnp.minimum(g, last)
        return idx // n_blk, idx % n_blk

    def emb_map(g):
        b, i = a_map(g)
        return (b, i, 0)

    def prev_map(g):
        b, i = a_map(g)
        return (b, jnp.maximum(i * (R // W) - 1, 0), 0)

    def out_map(g):
        idx = jnp.maximum(g - 1, 0)
        return (idx // n_blk, idx % n_blk, 0)

    out = pl.pallas_call(
        _make_body(n_blk, last),
        grid=(G,),
        in_specs=[
            pl.BlockSpec((1, R, E), emb_map),
            pl.BlockSpec((1, W, E), prev_map),
            pl.BlockSpec((D, E), lambda g: (0, 0)),
            pl.BlockSpec((1, D), lambda g: (0, 0)),
            pl.BlockSpec((D, E), lambda g: (0, 0)),
            pl.BlockSpec((1, D), lambda g: (0, 0)),
            pl.BlockSpec((E, D), lambda g: (0, 0)),
            pl.BlockSpec((1, E), lambda g: (0, 0)),
        ],
        out_specs=pl.BlockSpec((1, R, E), out_map),
        out_shape=jax.ShapeDtypeStruct((B, T, E), jnp.float32),
        scratch_shapes=[
            pltpu.VMEM((R, R + W), jnp.float32),
            pltpu.VMEM((R + W, D), jnp.float32),
            pltpu.VMEM((R, 1), jnp.float32),
        ],
        compiler_params=pltpu.CompilerParams(
            dimension_semantics=("arbitrary",),
        ),
    )(
        embeddings,
        embeddings,
        W_write,
        b_write.reshape(1, D),
        W_read_q,
        b_read_q.reshape(1, D),
        W_out,
        b_out.reshape(1, E),
    )
    return out


# speedup vs baseline: 1.0122x; 1.0122x over previous
import jax
import jax.numpy as jnp
from jax.experimental import pallas as pl
from jax.experimental.pallas import tpu as pltpu

BLOCK_R = 512
WINDOW = 64
NEG = -1e30
QSCALE = (128 ** -0.5) * 1.4426950408889634


def _dotT(a, w):
    return jax.lax.dot_general(
        a, w, (((1,), (1,)), ((), ())), preferred_element_type=jnp.float32
    )


def _make_body(n_blk, last):
    def _fused_body(emb_ref, prev_ref, ww_ref, bw_ref, wq_ref, bq_ref,
                    wo_ref, bo_ref, out_ref, ret_s):
        g = pl.program_id(0)

        out = _dotT(ret_s[...], wo_ref[...]) + bo_ref[...]
        out_ref[0] = out

        @pl.when((g - 1) % n_blk == 0)
        def _zero_t0():
            out_ref[0, 0:1, :] = jnp.zeros((1, out.shape[1]), jnp.float32)

        a_i = jnp.minimum(g, last) % n_blk
        e = emb_ref[0]
        ep = prev_ref[0]

        q = (_dotT(e, wq_ref[...]) + bq_ref[...]) * QSCALE
        k_cur = _dotT(e, ww_ref[...]) + bw_ref[...]
        k_prev = _dotT(ep, ww_ref[...]) + bw_ref[...]
        keys = jnp.concatenate([k_prev, k_cur], axis=0)

        sim = _dotT(q, keys)
        rows = jax.lax.broadcasted_iota(jnp.int32, sim.shape, 0)
        cols = jax.lax.broadcasted_iota(jnp.int32, sim.shape, 1)
        valid = (cols >= rows) & (cols <= rows + WINDOW - 1) & \
            ((cols >= WINDOW) | (a_i > 0))
        sim = jnp.where(valid, sim, NEG)

        p = jnp.exp2(sim)
        denom = jnp.sum(p, axis=1, keepdims=True)
        ret_s[...] = jax.lax.dot_general(
            p, keys, (((1,), (0,)), ((), ())),
            preferred_element_type=jnp.float32) / denom

    return _fused_body


def kernel(embeddings, W_write, b_write, W_read_q, b_read_q, W_out, b_out):
    B, T, E = embeddings.shape
    D = W_write.shape[0]
    R, W = BLOCK_R, WINDOW
    n_blk = T // R
    G = B * n_blk + 1
    last = B * n_blk - 1

    def a_map(g):
        idx = jnp.minimum(g, last)
        return idx // n_blk, idx % n_blk

    def emb_map(g):
        b, i = a_map(g)
        return (b, i, 0)

    def prev_map(g):
        b, i = a_map(g)
        return (b, jnp.maximum(i * (R // W) - 1, 0), 0)

    def out_map(g):
        idx = jnp.maximum(g - 1, 0)
        return (idx // n_blk, idx % n_blk, 0)

    out = pl.pallas_call(
        _make_body(n_blk, last),
        grid=(G,),
        in_specs=[
            pl.BlockSpec((1, R, E), emb_map),
            pl.BlockSpec((1, W, E), prev_map),
            pl.BlockSpec((D, E), lambda g: (0, 0)),
            pl.BlockSpec((1, D), lambda g: (0, 0)),
            pl.BlockSpec((D, E), lambda g: (0, 0)),
            pl.BlockSpec((1, D), lambda g: (0, 0)),
            pl.BlockSpec((E, D), lambda g: (0, 0)),
            pl.BlockSpec((1, E), lambda g: (0, 0)),
        ],
        out_specs=pl.BlockSpec((1, R, E), out_map),
        out_shape=jax.ShapeDtypeStruct((B, T, E), jnp.float32),
        scratch_shapes=[
            pltpu.VMEM((R, D), jnp.float32),
        ],
        compiler_params=pltpu.CompilerParams(
            dimension_semantics=("arbitrary",),
        ),
    )(
        embeddings,
        embeddings,
        W_write,
        b_write.reshape(1, D),
        W_read_q,
        b_read_q.reshape(1, D),
        W_out,
        b_out.reshape(1, E),
    )
    return out
